# Initial kernel scaffold; baseline (speedup 1.0000x reference)
import jax
import jax.numpy as jnp
from jax.experimental import pallas as pl


def kernel(boxes, scores):
    raise NotImplementedError("write your pallas kernel here")


# speedup vs baseline: 13.0065x; 13.0065x over previous
import jax
import jax.numpy as jnp
from jax.experimental import pallas as pl
from jax.experimental.pallas import tpu as pltpu

N = 20000
NPAD = 20480
KSEL = 1000
KPAD = 1024
NBLK = NPAD // KPAD
TOPO = 100
OPAD = 128
IOU_T = 0.5
NEG = -1e9
_HI = jax.lax.Precision.HIGHEST


def _excl_cumsum(x):
    acc = x
    shift = 1
    while shift < NPAD:
        acc = acc + jnp.concatenate(
            [jnp.zeros((1, shift), jnp.float32), acc[:, : NPAD - shift]], axis=1
        )
        shift *= 2
    return acc - x


def _nms_body(s_ref, data_ref, out_ref):
    s_flat = s_ref[0:1, :]
    si = jax.lax.bitcast_convert_type(s_flat, jnp.int32)

    def bit_body(t, v):
        cand = v | (jnp.int32(1) << (30 - t))
        cnt = jnp.sum((si >= cand).astype(jnp.int32))
        return jnp.where(cnt >= KSEL, cand, v)

    v = jax.lax.fori_loop(0, 31, bit_body, jnp.int32(0))
    c_gt = jnp.sum((si > v).astype(jnp.int32))
    n_tie = (KSEL - c_gt).astype(jnp.float32)

    tie = si == v
    tie_rank = _excl_cumsum(tie.astype(jnp.float32))
    selected = (si > v) | (tie & (tie_rank < n_tie))
    sel_f = selected.astype(jnp.float32)
    pos = _excl_cumsum(sel_f)

    iota_pi = jax.lax.broadcasted_iota(jnp.int32, (KPAD, 1), 0)
    iota_p = iota_pi.astype(jnp.float32)
    compact = jnp.zeros((KPAD, 8), jnp.float32)
    for b in range(NBLK):
        lo = b * KPAD
        pos_b = pos[:, lo : lo + KPAD]
        sel_b = sel_f[:, lo : lo + KPAD]
        onehot = jnp.where((pos_b == iota_p) & (sel_b > 0.0), 1.0, 0.0)
        d_b = data_ref[pl.ds(lo, KPAD), :]
        compact = compact + jnp.dot(
            onehot, d_b, preferred_element_type=jnp.float32, precision=_HI
        )

    x1 = compact[:, 0:1]
    y1 = compact[:, 1:2]
    x2 = compact[:, 2:3]
    y2 = compact[:, 3:4]
    sc = compact[:, 4:5]
    ix = compact[:, 5:6]

    compact_t = compact.T
    x1r = compact_t[0:1, :]
    y1r = compact_t[1:2, :]
    x2r = compact_t[2:3, :]
    y2r = compact_t[3:4, :]
    scr = compact_t[4:5, :]
    ixr = compact_t[5:6, :]

    ltx = jnp.maximum(x1, x1r)
    lty = jnp.maximum(y1, y1r)
    rbx = jnp.minimum(x2, x2r)
    rby = jnp.minimum(y2, y2r)
    inter = jnp.maximum(rbx - ltx, 0.0) * jnp.maximum(rby - lty, 0.0)
    area_c = (x2 - x1) * (y2 - y1)
    area_r = (x2r - x1r) * (y2r - y1r)
    iou = inter / jnp.maximum(area_c + area_r - inter, 1e-9)

    pt = (scr > sc) | ((scr == sc) & (ixr < ix))
    valid_r = jax.lax.broadcasted_iota(jnp.int32, (1, KPAD), 1) < KSEL
    valid_c = iota_pi < KSEL
    mt = jnp.where((iou > IOU_T) & pt & valid_r, 1.0, 0.0)
    ptf = jnp.where(pt & valid_r, 1.0, 0.0)

    keep0 = jnp.where(valid_c, 1.0, 0.0)

    def cond(st):
        _, changed, t = st
        return changed & (t < KPAD + 8)

    def body(st):
        keep, _, t = st
        supp = jnp.dot(mt, keep, preferred_element_type=jnp.float32)
        newk = jnp.where(valid_c & (supp == 0.0), 1.0, 0.0)
        return newk, jnp.any(newk != keep), t + 1

    keep, _, _ = jax.lax.while_loop(cond, body, (keep0, True, jnp.int32(0)))

    nkept = jnp.sum(keep)
    supv = jnp.where(valid_c & (keep == 0.0), 1.0, 0.0)
    rk = jnp.dot(ptf, keep, preferred_element_type=jnp.float32)
    rs = jnp.dot(ptf, supv, preferred_element_type=jnp.float32)
    rank_out = jnp.where(keep > 0.0, rk, nkept + rs)
    out_sc = jnp.where(keep > 0.0, sc, NEG)
    sel_o = valid_c & (rank_out < float(TOPO))
    rank_eff = jnp.where(sel_o, rank_out, 1e9)

    iota_o = jax.lax.broadcasted_iota(jnp.int32, (1, OPAD), 1).astype(jnp.float32)
    owt = jnp.where(rank_eff == iota_o, 1.0, 0.0)
    outdata = jnp.concatenate(
        [compact[:, 0:4], out_sc, jnp.zeros((KPAD, 3), jnp.float32)], axis=1
    )
    out = jax.lax.dot_general(
        owt, outdata, (((0,), (0,)), ((), ())),
        preferred_element_type=jnp.float32, precision=_HI,
    )
    out_ref[...] = out


def kernel(boxes, scores):
    idxf = jnp.arange(N, dtype=jnp.float32)
    data8 = jnp.concatenate(
        [boxes, scores[:, None], idxf[:, None], jnp.zeros((N, 2), jnp.float32)],
        axis=1,
    )
    data8 = jnp.pad(data8, ((0, NPAD - N), (0, 0)))
    s_pad = jnp.pad(scores, (0, NPAD - N), constant_values=-1.0).reshape(1, NPAD)
    out = pl.pallas_call(
        _nms_body,
        out_shape=jax.ShapeDtypeStruct((OPAD, 8), jnp.float32),
        in_specs=[
            pl.BlockSpec(memory_space=pltpu.VMEM),
            pl.BlockSpec(memory_space=pltpu.VMEM),
        ],
        out_specs=pl.BlockSpec(memory_space=pltpu.VMEM),
    )(s_pad, data8)
    return out[:TOPO, :5]


# speedup vs baseline: 16.4564x; 1.2652x over previous
import functools

import jax
import jax.numpy as jnp
from jax import lax
from jax.experimental import pallas as pl
from jax.experimental.pallas import tpu as pltpu
from jax.experimental.pallas import tpu_sc as plsc

N = 20000
NPAD = 20480
HALF = NPAD // 2
CHUNK = 640
NVREG = CHUNK // 16
KCORE = 1000
CAND = 2048
KSEL = 1000
KPAD = 1024
TOPO = 100
OPAD = 128
IOU_T = 0.5
NEG = -1e9
_HI = lax.Precision.HIGHEST


def _splat_sum(vec, nbits):
    total = jnp.zeros((16,), jnp.int32)
    for b in range(nbits):
        bit = plsc.all_reduce_population_count(
            (lax.shift_right_logical(vec, b) & 1) == 1)
        total = total + lax.shift_left(bit, b)
    return total


def _sc_body(sbits_hbm, data8_hbm, out_hbm, si_v, chunk_v, cn_v,
             vec16_v, cntst_v, shared_cnt, sem):
    c = lax.axis_index("c")
    s = lax.axis_index("s")
    base = c * HALF + s * CHUNK

    pltpu.sync_copy(sbits_hbm.at[pl.ds(base, CHUNK)], si_v)
    rowbase = c * (HALF // 16) + s * (CHUNK // 16)
    pltpu.sync_copy(data8_hbm.at[pl.ds(rowbase, CHUNK // 16)], chunk_v)

    rows16 = lax.broadcasted_iota(jnp.int32, (16,), 0)
    col0 = jnp.zeros((16,), jnp.int32)
    col1 = jnp.ones((16,), jnp.int32)
    zero16 = jnp.zeros((16,), jnp.int32)

    col2 = jnp.full((16,), 2, jnp.int32)

    def publish3(a_vec, b_vec, c_vec):
        vec16_v[...] = jnp.where(
            rows16 == 0, a_vec,
            jnp.where(rows16 == 1, b_vec, jnp.where(rows16 == 2, c_vec, 0)))
        pltpu.sync_copy(
            vec16_v, shared_cnt.at[pl.ds(pl.multiple_of(s * 16, 8), 16)])
        plsc.subcore_barrier()
        pltpu.sync_copy(shared_cnt, cntst_v)
        va = plsc.load_gather(cntst_v, [rows16 * 16])
        vb = plsc.load_gather(cntst_v, [rows16 * 16 + 1])
        vc = plsc.load_gather(cntst_v, [rows16 * 16 + 2])
        plsc.subcore_barrier()
        return va, vb, vc

    def count_ge(cand):
        acc = zero16
        for g in range(NVREG):
            x = si_v[pl.ds(g * 16, 16)]
            acc = acc + jnp.where(x >= cand, 1, 0)
        return _splat_sum(acc, 6)

    v = zero16
    for t in range(15):
        b_hi = jnp.int32(1) << (29 - 2 * t)
        b_lo = jnp.int32(1) << (28 - 2 * t)
        c12 = v | b_hi | b_lo
        c10 = v | b_hi
        c01 = v | b_lo
        va, vb, vc = publish3(count_ge(c12), count_ge(c10), count_ge(c01))
        t12 = _splat_sum(va, 10)
        t10 = _splat_sum(vb, 10)
        t01 = _splat_sum(vc, 10)
        v = jnp.where(
            t12 >= KCORE, c12,
            jnp.where(t10 >= KCORE, c10, jnp.where(t01 >= KCORE, c01, v)))

    acc_gt = zero16
    acc_tie = zero16
    for g in range(NVREG):
        x = si_v[pl.ds(g * 16, 16)]
        acc_gt = acc_gt + jnp.where(x > v, 1, 0)
        acc_tie = acc_tie + jnp.where(x == v, 1, 0)

    vgt, vtie, _ = publish3(_splat_sum(acc_gt, 6), _splat_sum(acc_tie, 6),
                            zero16)
    n_tie = KCORE - _splat_sum(vgt, 10)
    mask_lt = rows16 < s
    tie_before = _splat_sum(jnp.where(mask_lt, vtie, 0), 10)
    tie_excl = plsc.cumsum(vtie) - vtie
    taken = jnp.clip(n_tie - tie_excl, 0, vtie)
    selcnt = vgt + taken
    slot_base = c * KPAD + _splat_sum(jnp.where(mask_lt, selcnt, 0), 10)

    tie_run = zero16
    sel_run = zero16
    for g in range(NVREG):
        x = si_v[pl.ds(g * 16, 16)]
        gt = x > v
        tie = x == v
        tie_i = jnp.where(tie, 1, 0)
        t_excl = plsc.cumsum(tie_i) - tie_i
        grank = tie_before + tie_run + t_excl
        sel = gt | (tie & (grank < n_tie))
        sel_i = jnp.where(sel, 1, 0)
        s_excl = plsc.cumsum(sel_i) - sel_i
        lpos = sel_run + s_excl
        e_vec = rows16 + g * 16
        for col in range(8):
            cv = jnp.full((16,), col, jnp.int32)
            flat = e_vec * 8 + col
            i0 = lax.shift_right_logical(flat, 7)
            i1 = flat & 127
            vals = plsc.load_gather(chunk_v, [i0, i1])
            plsc.store_scatter(cn_v, [lpos * 8 + col], vals, mask=sel)
        tie_run = tie_run + plsc.all_reduce_population_count(tie)
        sel_run = sel_run + plsc.all_reduce_population_count(sel)

    nsel = jnp.max(sel_run)
    sbase = jnp.max(slot_base)

    def out_body(j, carry):
        pltpu.sync_copy(cn_v.at[pl.ds(pl.multiple_of(j * 8, 8), 8)],
                        out_hbm.at[pl.ds(pl.multiple_of((sbase + j) * 8, 8), 8)])
        return carry

    lax.fori_loop(0, nsel, out_body, jnp.int32(0))


def _sc_select(sbits_pad, data8):
    mesh = plsc.VectorSubcoreMesh(core_axis_name="c", subcore_axis_name="s")
    kfn = functools.partial(
        pl.kernel,
        mesh=mesh,
        compiler_params=pltpu.CompilerParams(needs_layout_passes=False),
        out_type=jax.ShapeDtypeStruct((CAND * 8,), jnp.float32),
        scratch_types=[
            pltpu.VMEM((CHUNK,), jnp.int32),
            pltpu.VMEM((CHUNK // 16, 128), jnp.float32),
            pltpu.VMEM((CHUNK * 8,), jnp.float32),
            pltpu.VMEM((16,), jnp.int32),
            pltpu.VMEM((256,), jnp.int32),
            pltpu.VMEM_SHARED((256,), jnp.int32),
            pltpu.SemaphoreType.DMA,
        ],
    )(_sc_body)
    return kfn(sbits_pad, data8)


def _excl_cumsum(x, width):
    acc = x
    shift = 1
    while shift < width:
        acc = acc + jnp.concatenate(
            [jnp.zeros((1, shift), jnp.float32), acc[:, : width - shift]],
            axis=1)
        shift *= 2
    return acc - x


def _tc_body(cand_ref, out_ref):
    raw = cand_ref[...]
    iota_c2 = lax.broadcasted_iota(jnp.int32, (CAND, 1), 0)
    valid2 = (iota_c2 < KCORE) | ((iota_c2 >= KPAD) & (iota_c2 < KPAD + KCORE))
    lane8 = lax.broadcasted_iota(jnp.int32, (1, 8), 1)
    defaults = jnp.where(lane8 == 4, -1.0, 0.0)
    data = jnp.where(valid2, raw, defaults)
    data_t = data.T

    s_row = data_t[4:5, :]
    si = lax.bitcast_convert_type(s_row, jnp.int32)

    def bit_body(t, v):
        cand = v | (jnp.int32(1) << (30 - t))
        cnt = jnp.sum((si >= cand).astype(jnp.int32))
        return jnp.where(cnt >= KSEL, cand, v)

    v = lax.fori_loop(0, 31, bit_body, jnp.int32(0))
    c_gt = jnp.sum((si > v).astype(jnp.int32))
    n_tie = (KSEL - c_gt).astype(jnp.float32)

    tie = si == v
    tie_rank = _excl_cumsum(tie.astype(jnp.float32), CAND)
    selected = (si > v) | (tie & (tie_rank < n_tie))
    sel_f = selected.astype(jnp.float32)
    pos = _excl_cumsum(sel_f, CAND)

    iota_pi = lax.broadcasted_iota(jnp.int32, (KPAD, 1), 0)
    iota_p = iota_pi.astype(jnp.float32)
    compact = jnp.zeros((KPAD, 8), jnp.float32)
    for b in range(CAND // KPAD):
        lo = b * KPAD
        pos_b = pos[:, lo : lo + KPAD]
        sel_b = sel_f[:, lo : lo + KPAD]
        onehot = jnp.where((pos_b == iota_p) & (sel_b > 0.0), 1.0, 0.0)
        compact = compact + jnp.dot(
            onehot, data[lo : lo + KPAD, :],
            preferred_element_type=jnp.float32, precision=_HI)

    x1 = compact[:, 0:1]
    y1 = compact[:, 1:2]
    x2 = compact[:, 2:3]
    y2 = compact[:, 3:4]
    sc = compact[:, 4:5]
    ix = compact[:, 5:6]

    compact_t = compact.T
    x1r = compact_t[0:1, :]
    y1r = compact_t[1:2, :]
    x2r = compact_t[2:3, :]
    y2r = compact_t[3:4, :]
    scr = compact_t[4:5, :]
    ixr = compact_t[5:6, :]

    ltx = jnp.maximum(x1, x1r)
    lty = jnp.maximum(y1, y1r)
    rbx = jnp.minimum(x2, x2r)
    rby = jnp.minimum(y2, y2r)
    inter = jnp.maximum(rbx - ltx, 0.0) * jnp.maximum(rby - lty, 0.0)
    area_c = (x2 - x1) * (y2 - y1)
    area_r = (x2r - x1r) * (y2r - y1r)
    iou = inter / jnp.maximum(area_c + area_r - inter, 1e-9)

    pt = (scr > sc) | ((scr == sc) & (ixr < ix))
    valid_r = lax.broadcasted_iota(jnp.int32, (1, KPAD), 1) < KSEL
    valid_c = iota_pi < KSEL
    mt = jnp.where((iou > IOU_T) & pt & valid_r, 1.0, 0.0)
    ptf = jnp.where(pt & valid_r, 1.0, 0.0)

    keep0 = jnp.where(valid_c, 1.0, 0.0)

    def cond(st):
        _, changed, t = st
        return changed & (t < KPAD + 8)

    def body(st):
        keep, _, t = st
        supp = jnp.dot(mt, keep, preferred_element_type=jnp.float32)
        newk = jnp.where(valid_c & (supp == 0.0), 1.0, 0.0)
        return newk, jnp.any(newk != keep), t + 1

    keep, _, _ = lax.while_loop(cond, body, (keep0, True, jnp.int32(0)))

    nkept = jnp.sum(keep)
    supv = jnp.where(valid_c & (keep == 0.0), 1.0, 0.0)
    rk = jnp.dot(ptf, keep, preferred_element_type=jnp.float32)
    rs = jnp.dot(ptf, supv, preferred_element_type=jnp.float32)
    rank_out = jnp.where(keep > 0.0, rk, nkept + rs)
    out_sc = jnp.where(keep > 0.0, sc, NEG)
    sel_o = valid_c & (rank_out < float(TOPO))
    rank_eff = jnp.where(sel_o, rank_out, 1e9)

    iota_o = lax.broadcasted_iota(jnp.int32, (1, OPAD), 1).astype(jnp.float32)
    owt = jnp.where(rank_eff == iota_o, 1.0, 0.0)
    outdata = jnp.concatenate(
        [compact[:, 0:4], out_sc, jnp.zeros((KPAD, 3), jnp.float32)], axis=1)
    out = lax.dot_general(
        owt, outdata, (((0,), (0,)), ((), ())),
        preferred_element_type=jnp.float32, precision=_HI)
    out_ref[...] = out


def kernel(boxes, scores):
    idxf = jnp.arange(N, dtype=jnp.float32)
    data8 = jnp.concatenate(
        [boxes, scores[:, None], idxf[:, None], jnp.zeros((N, 2), jnp.float32)],
        axis=1)
    data8 = jnp.pad(data8, ((0, NPAD - N), (0, 0))).reshape(NPAD // 16, 128)
    s_pad = jnp.pad(scores, (0, NPAD - N), constant_values=-1.0)
    sbits = lax.bitcast_convert_type(s_pad, jnp.int32)
    cand = _sc_select(sbits, data8).reshape(CAND, 8)
    out = pl.pallas_call(
        _tc_body,
        out_shape=jax.ShapeDtypeStruct((OPAD, 8), jnp.float32),
        in_specs=[pl.BlockSpec(memory_space=pltpu.VMEM)],
        out_specs=pl.BlockSpec(memory_space=pltpu.VMEM),
    )(cand)
    return out[:TOPO, :5]


# speedup vs baseline: 17.2877x; 1.0505x over previous
import functools

import jax
import jax.numpy as jnp
from jax import lax
from jax.experimental import pallas as pl
from jax.experimental.pallas import tpu as pltpu
from jax.experimental.pallas import tpu_sc as plsc

N = 20000
NPAD = 20480
HALF = NPAD // 2
CHUNK = 640
NVREG = CHUNK // 16
KCORE = 1000
CAND = 2048
KSEL = 1000
KPAD = 1024
TOPO = 100
OPAD = 128
IOU_T = 0.5
NEG = -1e9
_HI = lax.Precision.HIGHEST


def _splat_sum(vec, nbits):
    total = jnp.zeros((16,), jnp.int32)
    for b in range(nbits):
        bit = plsc.all_reduce_population_count(
            (lax.shift_right_logical(vec, b) & 1) == 1)
        total = total + lax.shift_left(bit, b)
    return total


def _sc_body(sbits_hbm, data8_hbm, out_hbm, si_v, chunk_v, cn_v,
             vec16_v, cntst_v, shared_cnt, sem):
    c = lax.axis_index("c")
    s = lax.axis_index("s")
    base = c * HALF + s * CHUNK

    pltpu.sync_copy(sbits_hbm.at[pl.ds(base, CHUNK)], si_v)
    rowbase = c * (HALF // 16) + s * (CHUNK // 16)
    pltpu.sync_copy(data8_hbm.at[pl.ds(rowbase, CHUNK // 16)], chunk_v)

    rows16 = lax.broadcasted_iota(jnp.int32, (16,), 0)
    col0 = jnp.zeros((16,), jnp.int32)
    col1 = jnp.ones((16,), jnp.int32)
    zero16 = jnp.zeros((16,), jnp.int32)

    col2 = jnp.full((16,), 2, jnp.int32)

    def publish3(a_vec, b_vec, c_vec):
        vec16_v[...] = jnp.where(
            rows16 == 0, a_vec,
            jnp.where(rows16 == 1, b_vec, jnp.where(rows16 == 2, c_vec, 0)))
        pltpu.sync_copy(
            vec16_v, shared_cnt.at[pl.ds(pl.multiple_of(s * 16, 8), 16)])
        plsc.subcore_barrier()
        pltpu.sync_copy(shared_cnt, cntst_v)
        va = plsc.load_gather(cntst_v, [rows16 * 16])
        vb = plsc.load_gather(cntst_v, [rows16 * 16 + 1])
        vc = plsc.load_gather(cntst_v, [rows16 * 16 + 2])
        plsc.subcore_barrier()
        return va, vb, vc

    def count_ge(cand):
        acc = zero16
        for g in range(NVREG):
            x = si_v[pl.ds(g * 16, 16)]
            acc = acc + jnp.where(x >= cand, 1, 0)
        return _splat_sum(acc, 6)

    v = zero16
    for t in range(15):
        b_hi = jnp.int32(1) << (29 - 2 * t)
        b_lo = jnp.int32(1) << (28 - 2 * t)
        c12 = v | b_hi | b_lo
        c10 = v | b_hi
        c01 = v | b_lo
        va, vb, vc = publish3(count_ge(c12), count_ge(c10), count_ge(c01))
        t12 = _splat_sum(va, 10)
        t10 = _splat_sum(vb, 10)
        t01 = _splat_sum(vc, 10)
        v = jnp.where(
            t12 >= KCORE, c12,
            jnp.where(t10 >= KCORE, c10, jnp.where(t01 >= KCORE, c01, v)))

    acc_gt = zero16
    acc_tie = zero16
    for g in range(NVREG):
        x = si_v[pl.ds(g * 16, 16)]
        acc_gt = acc_gt + jnp.where(x > v, 1, 0)
        acc_tie = acc_tie + jnp.where(x == v, 1, 0)

    vgt, vtie, _ = publish3(_splat_sum(acc_gt, 6), _splat_sum(acc_tie, 6),
                            zero16)
    n_tie = KCORE - _splat_sum(vgt, 10)
    mask_lt = rows16 < s
    tie_before = _splat_sum(jnp.where(mask_lt, vtie, 0), 10)
    tie_excl = plsc.cumsum(vtie) - vtie
    taken = jnp.clip(n_tie - tie_excl, 0, vtie)
    selcnt = vgt + taken
    slot_base = c * KPAD + _splat_sum(jnp.where(mask_lt, selcnt, 0), 10)

    tie_run = zero16
    sel_run = zero16
    for g in range(NVREG):
        x = si_v[pl.ds(g * 16, 16)]
        gt = x > v
        tie = x == v
        tie_i = jnp.where(tie, 1, 0)
        t_excl = plsc.cumsum(tie_i) - tie_i
        grank = tie_before + tie_run + t_excl
        sel = gt | (tie & (grank < n_tie))
        sel_i = jnp.where(sel, 1, 0)
        s_excl = plsc.cumsum(sel_i) - sel_i
        lpos = sel_run + s_excl
        e_vec = rows16 + g * 16
        for col in range(8):
            cv = jnp.full((16,), col, jnp.int32)
            flat = e_vec * 8 + col
            i0 = lax.shift_right_logical(flat, 7)
            i1 = flat & 127
            vals = plsc.load_gather(chunk_v, [i0, i1])
            plsc.store_scatter(cn_v, [lpos * 8 + col], vals, mask=sel)
        tie_run = tie_run + plsc.all_reduce_population_count(tie)
        sel_run = sel_run + plsc.all_reduce_population_count(sel)

    nsel = jnp.max(sel_run)
    sbase = jnp.max(slot_base)

    def out_body(j, carry):
        pltpu.sync_copy(cn_v.at[pl.ds(pl.multiple_of(j * 8, 8), 8)],
                        out_hbm.at[pl.ds(pl.multiple_of((sbase + j) * 8, 8), 8)])
        return carry

    lax.fori_loop(0, nsel, out_body, jnp.int32(0))


def _sc_select(sbits_pad, data8):
    mesh = plsc.VectorSubcoreMesh(core_axis_name="c", subcore_axis_name="s")
    kfn = functools.partial(
        pl.kernel,
        mesh=mesh,
        compiler_params=pltpu.CompilerParams(needs_layout_passes=False),
        out_type=jax.ShapeDtypeStruct((CAND * 8,), jnp.float32),
        scratch_types=[
            pltpu.VMEM((CHUNK,), jnp.int32),
            pltpu.VMEM((CHUNK // 16, 128), jnp.float32),
            pltpu.VMEM((CHUNK * 8,), jnp.float32),
            pltpu.VMEM((16,), jnp.int32),
            pltpu.VMEM((256,), jnp.int32),
            pltpu.VMEM_SHARED((256,), jnp.int32),
            pltpu.SemaphoreType.DMA,
        ],
    )(_sc_body)
    return kfn(sbits_pad, data8)


def _excl_cumsum(x, width):
    acc = x
    shift = 1
    while shift < width:
        acc = acc + jnp.concatenate(
            [jnp.zeros((1, shift), jnp.float32), acc[:, : width - shift]],
            axis=1)
        shift *= 2
    return acc - x


def _tc_body(cand_ref, out_ref):
    raw = cand_ref[...]
    iota_c2 = lax.broadcasted_iota(jnp.int32, (CAND, 1), 0)
    valid2 = (iota_c2 < KCORE) | ((iota_c2 >= KPAD) & (iota_c2 < KPAD + KCORE))
    lane8 = lax.broadcasted_iota(jnp.int32, (1, 8), 1)
    defaults = jnp.where(lane8 == 4, -1.0, 0.0)
    data = jnp.where(valid2, raw, defaults)
    data_t = data.T

    s_row = data_t[4:5, :]
    si = lax.bitcast_convert_type(s_row, jnp.int32)

    def bit_body(t, v):
        cand = v | (jnp.int32(1) << (30 - t))
        cnt = jnp.sum((si >= cand).astype(jnp.int32))
        return jnp.where(cnt >= KSEL, cand, v)

    v = lax.fori_loop(0, 31, bit_body, jnp.int32(0))
    c_gt = jnp.sum((si > v).astype(jnp.int32))
    n_tie = (KSEL - c_gt).astype(jnp.float32)

    tie = si == v
    tie_rank = _excl_cumsum(tie.astype(jnp.float32), CAND)
    selected = (si > v) | (tie & (tie_rank < n_tie))
    sel_f = selected.astype(jnp.float32)
    pos = _excl_cumsum(sel_f, CAND)

    iota_pi = lax.broadcasted_iota(jnp.int32, (KPAD, 1), 0)
    valid_c = iota_pi < KSEL
    valid_r = lax.broadcasted_iota(jnp.int32, (1, KPAD), 1) < KSEL
    iota_p_row = lax.broadcasted_iota(
        jnp.int32, (1, KPAD), 1).astype(jnp.float32)

    pos_col = pos.T
    sel_col = sel_f.T
    onehot_t = jnp.where((pos_col == iota_p_row) & (sel_col > 0.0), 1.0, 0.0)
    compact_t = jnp.dot(data_t, onehot_t,
                        preferred_element_type=jnp.float32, precision=_HI)
    compact = compact_t.T

    x1 = compact[:, 0:1]
    y1 = compact[:, 1:2]
    x2 = compact[:, 2:3]
    y2 = compact[:, 3:4]
    sc = compact[:, 4:5]
    ix = compact[:, 5:6]
    x1r = compact_t[0:1, :]
    y1r = compact_t[1:2, :]
    x2r = compact_t[2:3, :]
    y2r = compact_t[3:4, :]
    scr = compact_t[4:5, :]
    ixr = compact_t[5:6, :]

    ltx = jnp.maximum(x1, x1r)
    lty = jnp.maximum(y1, y1r)
    rbx = jnp.minimum(x2, x2r)
    rby = jnp.minimum(y2, y2r)
    inter = jnp.maximum(rbx - ltx, 0.0) * jnp.maximum(rby - lty, 0.0)
    area_c = (x2 - x1) * (y2 - y1)
    area_r = (x2r - x1r) * (y2r - y1r)
    iou = inter / jnp.maximum(area_c + area_r - inter, 1e-9)

    pbeat = (sc > scr) | ((sc == scr) & (ix < ixr))
    m2 = jnp.where((iou > IOU_T) & pbeat & valid_c, 1.0, 0.0)
    p2 = jnp.where(pbeat, 1.0, 0.0)

    keep0 = jnp.where(valid_r, 1.0, 0.0)

    def cond(st):
        _, changed, t = st
        return changed & (t < KPAD + 8)

    def body(st):
        keep, _, t = st
        supp = jnp.dot(keep, m2, preferred_element_type=jnp.float32)
        newk = jnp.where(valid_r & (supp == 0.0), 1.0, 0.0)
        return newk, jnp.any(newk != keep), t + 1

    keep, _, _ = lax.while_loop(cond, body, (keep0, True, jnp.int32(0)))

    nkept = jnp.sum(keep)
    supv = jnp.where(valid_r & (keep == 0.0), 1.0, 0.0)
    rk = jnp.dot(keep, p2, preferred_element_type=jnp.float32)
    rs = jnp.dot(supv, p2, preferred_element_type=jnp.float32)
    rank_out = jnp.where(keep > 0.0, rk, nkept + rs)
    out_sc_row = jnp.where(keep > 0.0, scr, NEG)
    sel_o = valid_r & (rank_out < float(TOPO))
    rank_eff = jnp.where(sel_o, rank_out, 1e9)

    iota_o_col = lax.broadcasted_iota(
        jnp.int32, (OPAD, 1), 0).astype(jnp.float32)
    owt = jnp.where(rank_eff == iota_o_col, 1.0, 0.0)
    outdata = jnp.concatenate(
        [compact[:, 0:4], out_sc_row.T, jnp.zeros((KPAD, 3), jnp.float32)],
        axis=1)
    out = jnp.dot(owt, outdata,
                  preferred_element_type=jnp.float32, precision=_HI)
    out_ref[...] = out


def kernel(boxes, scores):
    idxf = jnp.arange(N, dtype=jnp.float32)
    data8 = jnp.concatenate(
        [boxes, scores[:, None], idxf[:, None], jnp.zeros((N, 2), jnp.float32)],
        axis=1)
    data8 = jnp.pad(data8, ((0, NPAD - N), (0, 0))).reshape(NPAD // 16, 128)
    s_pad = jnp.pad(scores, (0, NPAD - N), constant_values=-1.0)
    sbits = lax.bitcast_convert_type(s_pad, jnp.int32)
    cand = _sc_select(sbits, data8).reshape(CAND, 8)
    out = pl.pallas_call(
        _tc_body,
        out_shape=jax.ShapeDtypeStruct((OPAD, 8), jnp.float32),
        in_specs=[pl.BlockSpec(memory_space=pltpu.VMEM)],
        out_specs=pl.BlockSpec(memory_space=pltpu.VMEM),
    )(cand)
    return out[:TOPO, :5]


# speedup vs baseline: 17.5745x; 1.0166x over previous
import functools

import jax
import jax.numpy as jnp
from jax import lax
from jax.experimental import pallas as pl
from jax.experimental.pallas import tpu as pltpu
from jax.experimental.pallas import tpu_sc as plsc

N = 20000
NPAD = 20480
HALF = NPAD // 2
CHUNK = 640
NVREG = CHUNK // 16
KCORE = 1000
CAND = 2048
KSEL = 1000
KPAD = 1024
TOPO = 100
OPAD = 128
IOU_T = 0.5
NEG = -1e9
_HI = lax.Precision.HIGHEST


def _splat_sum(vec, nbits):
    total = jnp.zeros((16,), jnp.int32)
    for b in range(nbits):
        bit = plsc.all_reduce_population_count(
            (lax.shift_right_logical(vec, b) & 1) == 1)
        total = total + lax.shift_left(bit, b)
    return total


def _sc_body(sbits_hbm, data8_hbm, out_hbm, si_v, chunk_v, cn_v,
             vec16_v, cntst_v, shared_cnt, sem):
    c = lax.axis_index("c")
    s = lax.axis_index("s")
    base = c * HALF + s * CHUNK

    pltpu.sync_copy(sbits_hbm.at[pl.ds(base, CHUNK)], si_v)
    rowbase = c * (HALF // 16) + s * (CHUNK // 16)
    chunk_cp = pltpu.async_copy(
        data8_hbm.at[pl.ds(rowbase, CHUNK // 16)], chunk_v, sem)

    rows16 = lax.broadcasted_iota(jnp.int32, (16,), 0)
    col0 = jnp.zeros((16,), jnp.int32)
    col1 = jnp.ones((16,), jnp.int32)
    zero16 = jnp.zeros((16,), jnp.int32)

    col2 = jnp.full((16,), 2, jnp.int32)

    def publish3(rnd, a_vec, b_vec, c_vec):
        vec16_v[...] = jnp.where(
            rows16 == 0, a_vec,
            jnp.where(rows16 == 1, b_vec, jnp.where(rows16 == 2, c_vec, 0)))
        pltpu.sync_copy(
            vec16_v,
            shared_cnt.at[pl.ds(pl.multiple_of(rnd * 256 + s * 16, 8), 16)])
        plsc.subcore_barrier()
        pltpu.sync_copy(shared_cnt.at[pl.ds(rnd * 256, 256)], cntst_v)
        va = plsc.load_gather(cntst_v, [rows16 * 16])
        vb = plsc.load_gather(cntst_v, [rows16 * 16 + 1])
        vc = plsc.load_gather(cntst_v, [rows16 * 16 + 2])
        return va, vb, vc

    def count_ge(cand):
        acc = zero16
        for g in range(NVREG):
            x = si_v[pl.ds(g * 16, 16)]
            acc = acc + jnp.where(x >= cand, 1, 0)
        return _splat_sum(acc, 6)

    v = zero16
    for t in range(15):
        b_hi = jnp.int32(1) << (29 - 2 * t)
        b_lo = jnp.int32(1) << (28 - 2 * t)
        c12 = v | b_hi | b_lo
        c10 = v | b_hi
        c01 = v | b_lo
        va, vb, vc = publish3(t, count_ge(c12), count_ge(c10), count_ge(c01))
        t12 = _splat_sum(va, 10)
        t10 = _splat_sum(vb, 10)
        t01 = _splat_sum(vc, 10)
        v = jnp.where(
            t12 >= KCORE, c12,
            jnp.where(t10 >= KCORE, c10, jnp.where(t01 >= KCORE, c01, v)))

    acc_gt = zero16
    acc_tie = zero16
    for g in range(NVREG):
        x = si_v[pl.ds(g * 16, 16)]
        acc_gt = acc_gt + jnp.where(x > v, 1, 0)
        acc_tie = acc_tie + jnp.where(x == v, 1, 0)

    vgt, vtie, _ = publish3(15, _splat_sum(acc_gt, 6), _splat_sum(acc_tie, 6),
                            zero16)
    n_tie = KCORE - _splat_sum(vgt, 10)
    mask_lt = rows16 < s
    tie_before = _splat_sum(jnp.where(mask_lt, vtie, 0), 10)
    tie_excl = plsc.cumsum(vtie) - vtie
    taken = jnp.clip(n_tie - tie_excl, 0, vtie)
    selcnt = vgt + taken
    slot_base = c * KPAD + _splat_sum(jnp.where(mask_lt, selcnt, 0), 10)

    chunk_cp.wait()
    tie_run = zero16
    sel_run = zero16
    for g in range(NVREG):
        x = si_v[pl.ds(g * 16, 16)]
        gt = x > v
        tie = x == v
        tie_i = jnp.where(tie, 1, 0)
        t_excl = plsc.cumsum(tie_i) - tie_i
        grank = tie_before + tie_run + t_excl
        sel = gt | (tie & (grank < n_tie))
        sel_i = jnp.where(sel, 1, 0)
        s_excl = plsc.cumsum(sel_i) - sel_i
        lpos = sel_run + s_excl
        e_vec = rows16 + g * 16
        for col in range(8):
            cv = jnp.full((16,), col, jnp.int32)
            flat = e_vec * 8 + col
            i0 = lax.shift_right_logical(flat, 7)
            i1 = flat & 127
            vals = plsc.load_gather(chunk_v, [i0, i1])
            plsc.store_scatter(cn_v, [lpos * 8 + col], vals, mask=sel)
        tie_run = tie_run + plsc.all_reduce_population_count(tie)
        sel_run = sel_run + plsc.all_reduce_population_count(sel)

    nsel = jnp.max(sel_run)
    sbase = jnp.max(slot_base)

    def out_body(j, carry):
        pltpu.sync_copy(cn_v.at[pl.ds(pl.multiple_of(j * 8, 8), 8)],
                        out_hbm.at[pl.ds(pl.multiple_of((sbase + j) * 8, 8), 8)])
        return carry

    lax.fori_loop(0, nsel, out_body, jnp.int32(0))


def _sc_select(sbits_pad, data8):
    mesh = plsc.VectorSubcoreMesh(core_axis_name="c", subcore_axis_name="s")
    kfn = functools.partial(
        pl.kernel,
        mesh=mesh,
        compiler_params=pltpu.CompilerParams(needs_layout_passes=False),
        out_type=jax.ShapeDtypeStruct((CAND * 8,), jnp.float32),
        scratch_types=[
            pltpu.VMEM((CHUNK,), jnp.int32),
            pltpu.VMEM((CHUNK // 16, 128), jnp.float32),
            pltpu.VMEM((CHUNK * 8,), jnp.float32),
            pltpu.VMEM((16,), jnp.int32),
            pltpu.VMEM((256,), jnp.int32),
            pltpu.VMEM_SHARED((4096,), jnp.int32),
            pltpu.SemaphoreType.DMA,
        ],
    )(_sc_body)
    return kfn(sbits_pad, data8)


def _excl_cumsum(x, width):
    acc = x
    shift = 1
    while shift < width:
        acc = acc + jnp.concatenate(
            [jnp.zeros((1, shift), jnp.float32), acc[:, : width - shift]],
            axis=1)
        shift *= 2
    return acc - x


def _tc_body(cand_ref, out_ref):
    raw = cand_ref[...]
    iota_c2 = lax.broadcasted_iota(jnp.int32, (CAND, 1), 0)
    valid2 = (iota_c2 < KCORE) | ((iota_c2 >= KPAD) & (iota_c2 < KPAD + KCORE))
    lane8 = lax.broadcasted_iota(jnp.int32, (1, 8), 1)
    defaults = jnp.where(lane8 == 4, -1.0, 0.0)
    data = jnp.where(valid2, raw, defaults)
    data_t = data.T

    s_row = data_t[4:5, :]
    si = lax.bitcast_convert_type(s_row, jnp.int32)

    def bit_body(t, v):
        cand = v | (jnp.int32(1) << (30 - t))
        cnt = jnp.sum((si >= cand).astype(jnp.int32))
        return jnp.where(cnt >= KSEL, cand, v)

    v = lax.fori_loop(0, 31, bit_body, jnp.int32(0))
    c_gt = jnp.sum((si > v).astype(jnp.int32))
    n_tie = (KSEL - c_gt).astype(jnp.float32)

    tie = si == v
    tie_rank = _excl_cumsum(tie.astype(jnp.float32), CAND)
    selected = (si > v) | (tie & (tie_rank < n_tie))
    sel_f = selected.astype(jnp.float32)
    pos = _excl_cumsum(sel_f, CAND)

    iota_pi = lax.broadcasted_iota(jnp.int32, (KPAD, 1), 0)
    valid_c = iota_pi < KSEL
    valid_r = lax.broadcasted_iota(jnp.int32, (1, KPAD), 1) < KSEL
    iota_p_row = lax.broadcasted_iota(
        jnp.int32, (1, KPAD), 1).astype(jnp.float32)

    pos_col = pos.T
    sel_col = sel_f.T
    onehot_t = jnp.where((pos_col == iota_p_row) & (sel_col > 0.0), 1.0, 0.0)
    compact_t = jnp.dot(data_t, onehot_t,
                        preferred_element_type=jnp.float32, precision=_HI)
    compact = compact_t.T

    x1 = compact[:, 0:1]
    y1 = compact[:, 1:2]
    x2 = compact[:, 2:3]
    y2 = compact[:, 3:4]
    sc = compact[:, 4:5]
    ix = compact[:, 5:6]
    x1r = compact_t[0:1, :]
    y1r = compact_t[1:2, :]
    x2r = compact_t[2:3, :]
    y2r = compact_t[3:4, :]
    scr = compact_t[4:5, :]
    ixr = compact_t[5:6, :]

    ltx = jnp.maximum(x1, x1r)
    lty = jnp.maximum(y1, y1r)
    rbx = jnp.minimum(x2, x2r)
    rby = jnp.minimum(y2, y2r)
    inter = jnp.maximum(rbx - ltx, 0.0) * jnp.maximum(rby - lty, 0.0)
    area_c = (x2 - x1) * (y2 - y1)
    area_r = (x2r - x1r) * (y2r - y1r)
    iou = inter / jnp.maximum(area_c + area_r - inter, 1e-9)

    pbeat = (sc > scr) | ((sc == scr) & (ix < ixr))
    m2 = jnp.where((iou > IOU_T) & pbeat & valid_c, 1.0, 0.0)
    p2 = jnp.where(pbeat, 1.0, 0.0)

    keep0 = jnp.where(valid_r, 1.0, 0.0)

    def cond(st):
        _, changed, t = st
        return changed & (t < KPAD + 8)

    def body(st):
        keep, _, t = st
        supp = jnp.dot(keep, m2, preferred_element_type=jnp.float32)
        newk = jnp.where(valid_r & (supp == 0.0), 1.0, 0.0)
        return newk, jnp.any(newk != keep), t + 1

    keep, _, _ = lax.while_loop(cond, body, (keep0, True, jnp.int32(0)))

    nkept = jnp.sum(keep)
    supv = jnp.where(valid_r & (keep == 0.0), 1.0, 0.0)
    rk = jnp.dot(keep, p2, preferred_element_type=jnp.float32)
    rs = jnp.dot(supv, p2, preferred_element_type=jnp.float32)
    rank_out = jnp.where(keep > 0.0, rk, nkept + rs)
    out_sc_row = jnp.where(keep > 0.0, scr, NEG)
    sel_o = valid_r & (rank_out < float(TOPO))
    rank_eff = jnp.where(sel_o, rank_out, 1e9)

    iota_o_col = lax.broadcasted_iota(
        jnp.int32, (OPAD, 1), 0).astype(jnp.float32)
    owt = jnp.where(rank_eff == iota_o_col, 1.0, 0.0)
    outdata = jnp.concatenate(
        [compact[:, 0:4], out_sc_row.T, jnp.zeros((KPAD, 3), jnp.float32)],
        axis=1)
    out = jnp.dot(owt, outdata,
                  preferred_element_type=jnp.float32, precision=_HI)
    out_ref[...] = out[:TOPO, :5]


def kernel(boxes, scores):
    idxf = jnp.arange(N, dtype=jnp.float32)
    data8 = jnp.concatenate(
        [boxes, scores[:, None], idxf[:, None], jnp.zeros((N, 2), jnp.float32)],
        axis=1)
    data8 = jnp.pad(data8, ((0, NPAD - N), (0, 0))).reshape(NPAD // 16, 128)
    s_pad = jnp.pad(scores, (0, NPAD - N), constant_values=-1.0)
    sbits = lax.bitcast_convert_type(s_pad, jnp.int32)
    cand = _sc_select(sbits, data8).reshape(CAND, 8)
    out = pl.pallas_call(
        _tc_body,
        out_shape=jax.ShapeDtypeStruct((TOPO, 5), jnp.float32),
        in_specs=[pl.BlockSpec(memory_space=pltpu.VMEM)],
        out_specs=pl.BlockSpec(memory_space=pltpu.VMEM),
    )(cand)
    return out
